# baseline (speedup 1.0000x reference)
import functools

import jax
import jax.numpy as jnp
from jax import lax
from jax.experimental import pallas as pl
from jax.experimental.pallas import tpu as pltpu
from jax.experimental.pallas import tpu_sc as plsc

N = 50000
E = 800000
D = 128
Q = 12544
NPAD = 4 * Q
STRIPE = Q // 16
BLK = 128
NBLK = E // BLK


def _spmv_body(with_deg, *refs):
    if with_deg:
        (tab, src_h, dst_h, w_h,
         out, out_deg,
         gsrc, gdl, gw, degval, rows,
         z128, zq, degb, acc_sh, deg_sh, sem) = refs
    else:
        (tab, src_h, dst_h, w_h,
         out,
         gsrc, gdl, gw, rows,
         z128, acc_sh, sem) = refs
        out_deg = deg_sh = zq = degval = None

    c = lax.axis_index("c")
    s = lax.axis_index("s")
    r0 = s * STRIPE
    zvf = jnp.zeros((16,), jnp.float32)

    def zfill(i, _):
        for j in range(8):
            z128[i, pl.ds(j * 16, 16)] = zvf
        return 0
    lax.fori_loop(0, 49, zfill, 0)
    if with_deg:
        def zfillq(i, _):
            zq[pl.ds(i * 16, 16)] = zvf
            return 0
        lax.fori_loop(0, STRIPE // 16, zfillq, 0)

    nblk_mine = (NBLK - s + 15) // 16

    for qi in range(2):
        q = 2 * c + qi
        base = q * Q
        for k in range(16):
            pltpu.sync_copy(z128, acc_sh.at[pl.ds(r0 + k * 49, 49)])
        if with_deg:
            pltpu.sync_copy(zq, deg_sh.at[pl.ds(r0, STRIPE)])
        plsc.subcore_barrier()

        def blk_body(i, _):
            b = s + i * 16
            e0 = b * BLK
            pltpu.sync_copy(src_h.at[pl.ds(e0, BLK)], gsrc)
            pltpu.sync_copy(dst_h.at[pl.ds(e0, BLK)], gdl)
            pltpu.sync_copy(w_h.at[pl.ds(e0, BLK)], gw)
            for g in range(8):
                sl = pl.ds(g * 16, 16)
                dl = gdl[sl] - base
                inr = (dl >= 0) & (dl < Q)
                gdl[sl] = jnp.minimum(jnp.maximum(dl, 0), Q - 1)
                gw[sl] = jnp.where(inr, gw[sl], zvf)
                if with_deg:
                    degval[sl] = jnp.where(
                        inr, jnp.full((16,), 1.0, jnp.float32), zvf)
            pltpu.async_copy(tab.at[gsrc], rows, sem).wait()

            def scale(g2, _):
                wg = gw[pl.ds(g2 * 16, 16)]
                for t in range(16):
                    e = g2 * 16 + t
                    wsc = wg[t]
                    for j in range(8):
                        sl2 = pl.ds(j * 16, 16)
                        rows[e, sl2] = rows[e, sl2] * wsc
                return 0
            lax.fori_loop(0, BLK // 16, scale, 0)

            pltpu.sync_copy(rows, acc_sh.at[gdl], add=True)
            if with_deg:
                pltpu.sync_copy(degval, deg_sh.at[gdl], add=True)
            return 0

        lax.fori_loop(0, nblk_mine, blk_body, 0)

        plsc.subcore_barrier()
        pltpu.sync_copy(acc_sh.at[pl.ds(r0, STRIPE)],
                        out.at[pl.ds(base + r0, STRIPE)])
        if with_deg:
            pltpu.sync_copy(deg_sh.at[pl.ds(r0, STRIPE)], degb)
            pltpu.sync_copy(degb, out_deg.at[pl.ds(base + r0, STRIPE)])
        plsc.subcore_barrier()


def _make_spmv(with_deg):
    mesh = plsc.VectorSubcoreMesh(core_axis_name="c", subcore_axis_name="s",
                                  num_cores=2, num_subcores=16)
    out_type = [jax.ShapeDtypeStruct((NPAD, D), jnp.float32)]
    if with_deg:
        out_type.append(jax.ShapeDtypeStruct((NPAD,), jnp.float32))
    scratch = [
        pltpu.VMEM((BLK,), jnp.int32),
        pltpu.VMEM((BLK,), jnp.int32),
        pltpu.VMEM((BLK,), jnp.float32),
    ]
    if with_deg:
        scratch.append(pltpu.VMEM((BLK,), jnp.float32))
    scratch.append(pltpu.VMEM((BLK, D), jnp.float32))
    scratch.append(pltpu.VMEM((49, D), jnp.float32))
    if with_deg:
        scratch.append(pltpu.VMEM((STRIPE,), jnp.float32))
        scratch.append(pltpu.VMEM((STRIPE,), jnp.float32))
    scratch.append(pltpu.VMEM_SHARED((Q, D), jnp.float32))
    if with_deg:
        scratch.append(pltpu.VMEM_SHARED((Q,), jnp.float32))
    scratch.append(pltpu.SemaphoreType.DMA)

    return pl.kernel(
        functools.partial(_spmv_body, with_deg),
        out_type=tuple(out_type) if with_deg else out_type[0],
        mesh=mesh,
        scratch_types=scratch,
    )


BR = 2000


def _combine2_body(relu, mlp, x, n1, d1, n2, d2, *rest):
    if mlp:
        (wsc, wn1, wn2, bb,
         l1w, l1b, l2w, l2b, l3w, l3b, out) = rest
    else:
        wsc, wn1, wn2, bb, out = rest
    r1 = 1.0 / jnp.maximum(d1[...], 1.0)
    r2 = 1.0 / jnp.maximum(d2[...], 1.0)
    acc = jnp.dot(x[...], wsc[...], preferred_element_type=jnp.float32)
    acc += jnp.dot(n1[...] * r1, wn1[...], preferred_element_type=jnp.float32)
    acc += jnp.dot(n2[...] * r2, wn2[...], preferred_element_type=jnp.float32)
    acc += bb[...]
    if relu:
        acc = jnp.maximum(acc, 0.0)
    if mlp:
        h = jnp.maximum(jnp.dot(acc, l1w[...],
                                preferred_element_type=jnp.float32) + l1b[...],
                        0.0)
        h = jnp.maximum(jnp.dot(h, l2w[...],
                                preferred_element_type=jnp.float32) + l2b[...],
                        0.0)
        out[...] = jnp.dot(h, l3w[...],
                           preferred_element_type=jnp.float32) + l3b[...]
    else:
        out[...] = acc


def _combine1_body(relu, x, nn, d, wsc, wn, bb, out):
    r = 1.0 / jnp.maximum(d[...], 1.0)
    acc = jnp.dot(x[...], wsc[...], preferred_element_type=jnp.float32)
    acc += jnp.dot(nn[...] * r, wn[...], preferred_element_type=jnp.float32)
    acc += bb[...]
    if relu:
        acc = jnp.maximum(acc, 0.0)
    out[...] = acc


def _row_spec(w):
    return pl.BlockSpec((BR, w), lambda i: (i, 0))


def _whole_spec(shape):
    return pl.BlockSpec(shape, lambda i: tuple(0 for _ in shape))


def _combine2(relu, mlp, x, n1, d1, n2, d2, wsc, wn1, wn2, bb, lws=None):
    grid = N // BR
    in_specs = [_row_spec(D), _row_spec(D), _row_spec(1),
                _row_spec(D), _row_spec(1),
                _whole_spec((D, D)), _whole_spec((D, D)), _whole_spec((D, D)),
                _whole_spec((1, D))]
    args = [x, n1, d1, n2, d2, wsc, wn1, wn2, bb]
    if mlp:
        l1w, l1b, l2w, l2b, l3w, l3b = lws
        in_specs += [_whole_spec((D, 64)), _whole_spec((1, 64)),
                     _whole_spec((64, 32)), _whole_spec((1, 32)),
                     _whole_spec((32, 4)), _whole_spec((1, 4))]
        args += [l1w, l1b, l2w, l2b, l3w, l3b]
        out_shape = jax.ShapeDtypeStruct((N, 4), jnp.float32)
        out_specs = _row_spec(4)
    else:
        out_shape = jax.ShapeDtypeStruct((N, D), jnp.float32)
        out_specs = _row_spec(D)
    return pl.pallas_call(
        functools.partial(_combine2_body, relu, mlp),
        grid=(grid,),
        in_specs=in_specs,
        out_specs=out_specs,
        out_shape=out_shape,
    )(*args)


def _combine1(relu, x, nn, d, wsc, wn, bb):
    grid = N // BR
    in_specs = [_row_spec(D), _row_spec(D), _row_spec(1),
                _whole_spec((D, D)), _whole_spec((D, D)), _whole_spec((1, D))]
    return pl.pallas_call(
        functools.partial(_combine1_body, relu),
        grid=(grid,),
        in_specs=in_specs,
        out_specs=_row_spec(D),
        out_shape=jax.ShapeDtypeStruct((N, D), jnp.float32),
    )(x, nn, d, wsc, wn, bb)


def kernel(x_acoustic, x_word, ei_sim_tic, ei_sim_w, ei_related_to,
           ew_sim_tic, ew_sim_w, ew_related_to,
           c1_tic_Ws, c1_tic_Wn, c1_tic_b, c1_w_Ws, c1_w_Wn, c1_w_b,
           c1_rel_Ws, c1_rel_Wn, c1_rel_b,
           c2_tic_Ws, c2_tic_Wn, c2_tic_b, c2_w_Ws, c2_w_Wn, c2_w_b,
           c2_rel_Ws, c2_rel_Wn, c2_rel_b,
           l1_W, l1_b, l2_W, l2_b, l3_W, l3_b):
    spmv_deg = _make_spmv(True)
    spmv = _make_spmv(False)

    src_t, dst_t = ei_sim_tic[0], ei_sim_tic[1]
    src_r, dst_r = ei_related_to[0], ei_related_to[1]
    src_w, dst_w = ei_sim_w[0], ei_sim_w[1]

    a1t, degt = spmv_deg(x_acoustic, src_t, dst_t, ew_sim_tic)
    a1r, degr = spmv_deg(x_acoustic, src_r, dst_r, ew_related_to)
    a1w, degw = spmv_deg(x_word, src_w, dst_w, ew_sim_w)

    d_t = degt[:N, None]
    d_r = degr[:N, None]
    d_w = degw[:N, None]

    wsc1 = (c1_tic_Ws + c1_rel_Ws) * 0.5
    b1 = ((c1_tic_b + c1_rel_b) * 0.5)[None, :]
    a_full = _combine2(True, False,
                       x_acoustic, a1t[:N], d_t, a1r[:N], d_r,
                       wsc1, c1_tic_Wn * 0.5, c1_rel_Wn * 0.5, b1)

    w_full = _combine1(True, x_word, a1w[:N], d_w,
                       c1_w_Ws, c1_w_Wn, c1_w_b[None, :])

    a2t = spmv(a_full, src_t, dst_t, ew_sim_tic)
    a2r = spmv(a_full, src_r, dst_r, ew_related_to)
    a2w = spmv(w_full, src_w, dst_w, ew_sim_w)

    wsc2 = (c2_tic_Ws + c2_rel_Ws) * 0.5
    b2 = ((c2_tic_b + c2_rel_b) * 0.5)[None, :]
    h = _combine2(False, True,
                  a_full, a2t[:N], d_t, a2r[:N], d_r,
                  wsc2, c2_tic_Wn * 0.5, c2_rel_Wn * 0.5, b2,
                  lws=(l1_W, l1_b[None, :], l2_W, l2_b[None, :],
                       l3_W, l3_b[None, :]))

    w2 = _combine1(False, w_full, a2w[:N], d_w,
                   c2_w_Ws, c2_w_Wn, c2_w_b[None, :])

    return h, h, w2


# speedup vs baseline: 1.2346x; 1.2346x over previous
import functools

import jax
import jax.numpy as jnp
from jax import lax
from jax.experimental import pallas as pl
from jax.experimental.pallas import tpu as pltpu
from jax.experimental.pallas import tpu_sc as plsc

N = 50000
E = 800000
D = 128
Q = 12544
NPAD = 4 * Q
STRIPE = Q // 16
BLK = 80
NBLK = E // BLK
MW = 2 * BLK


def _spmv_body(with_deg, *refs):
    if with_deg:
        (tab, meta_h, w_h,
         out, out_deg,
         pmA, pmB, wbA, wbB, gdlA, gdlB, gwA, gwB, dvA, dvB, rowsA, rowsB,
         z128, zq, degb, acc_sh, deg_sh,
         semGA, semGB, semSA, semSB, semDA, semDB) = refs
    else:
        (tab, meta_h, w_h,
         out,
         pmA, pmB, wbA, wbB, gdlA, gdlB, gwA, gwB, rowsA, rowsB,
         z128, acc_sh,
         semGA, semGB, semSA, semSB) = refs
        out_deg = deg_sh = zq = degb = dvA = dvB = None
        semDA = semDB = None

    c = lax.axis_index("c")
    s = lax.axis_index("s")
    r0 = s * STRIPE
    zvf = jnp.zeros((16,), jnp.float32)

    def zfill(i, _):
        for j in range(8):
            z128[i, pl.ds(j * 16, 16)] = zvf
        return 0
    lax.fori_loop(0, 16, zfill, 0)
    if with_deg:
        def zfillq(i, _):
            zq[pl.ds(i * 16, 16)] = zvf
            return 0
        lax.fori_loop(0, STRIPE // 16, zfillq, 0)

    npairs = (NBLK // 16) // 2

    def masks(pm, wb, gdlb, gwb, dvb, base):
        for g in range(8):
            if g * 16 >= BLK:
                break
            sl = pl.ds(g * 16, 16)
            dl = pm[pl.ds(BLK + g * 16, 16)] - base
            inr = (dl >= 0) & (dl < Q)
            gdlb[sl] = jnp.minimum(jnp.maximum(dl, 0), Q - 1)
            gwb[sl] = jnp.where(inr, wb[sl], zvf)
            if with_deg:
                dvb[sl] = jnp.where(
                    inr, jnp.full((16,), 1.0, jnp.float32), zvf)

    def scale(rows, gwb):
        def sc(g2, _):
            wg = gwb[pl.ds(g2 * 16, 16)]
            for t in range(16):
                e = g2 * 16 + t
                wsc = wg[t]
                for j in range(8):
                    sl2 = pl.ds(j * 16, 16)
                    rows[e, sl2] = rows[e, sl2] * wsc
            return 0
        lax.fori_loop(0, BLK // 16, sc, 0)

    for qi in range(2):
        q = 2 * c + qi
        base = q * Q
        for k in range(49):
            pltpu.sync_copy(z128, acc_sh.at[pl.ds(r0 + k * 16, 16)])
        if with_deg:
            pltpu.sync_copy(zq, deg_sh.at[pl.ds(r0, STRIPE)])
        plsc.subcore_barrier()

        def blk_pair(i, _):
            jA = 2 * i
            jB = 2 * i + 1
            bA = s + jA * 16
            bB = s + jB * 16
            validB = jB < (NBLK // 16)
            pltpu.sync_copy(meta_h.at[pl.ds(bA * MW, MW)], pmA)
            pltpu.sync_copy(w_h.at[pl.ds(bA * BLK, BLK)], wbA)
            gA = pltpu.async_copy(tab.at[pmA.at[pl.ds(0, BLK)]], rowsA,
                                  semGA)

            @pl.when(validB)
            def _():
                pltpu.sync_copy(meta_h.at[pl.ds(bB * MW, MW)], pmB)
                pltpu.sync_copy(w_h.at[pl.ds(bB * BLK, BLK)], wbB)
                pltpu.async_copy(tab.at[pmB.at[pl.ds(0, BLK)]], rowsB,
                                 semGB)

            masks(pmA, wbA, gdlA, gwA, dvA, base)

            @pl.when(validB)
            def _():
                masks(pmB, wbB, gdlB, gwB, dvB, base)

            gA.wait()
            scale(rowsA, gwA)
            sA = pltpu.async_copy(rowsA, acc_sh.at[gdlA], semSA, add=True)
            if with_deg:
                dA = pltpu.async_copy(dvA, deg_sh.at[gdlA], semDA, add=True)

            @pl.when(validB)
            def _():
                pltpu.make_async_copy(tab.at[pmB.at[pl.ds(0, BLK)]], rowsB,
                                      semGB).wait()
                scale(rowsB, gwB)
                pltpu.async_copy(rowsB, acc_sh.at[gdlB], semSB, add=True)
                if with_deg:
                    pltpu.async_copy(dvB, deg_sh.at[gdlB], semDB, add=True)

            sA.wait()
            if with_deg:
                dA.wait()

            @pl.when(validB)
            def _():
                pltpu.make_async_copy(rowsB, acc_sh.at[gdlB], semSB).wait()
                if with_deg:
                    pltpu.make_async_copy(dvB, deg_sh.at[gdlB], semDB).wait()
            return 0

        lax.fori_loop(0, npairs + 1, blk_pair, 0)

        plsc.subcore_barrier()
        pltpu.sync_copy(acc_sh.at[pl.ds(r0, STRIPE)],
                        out.at[pl.ds(base + r0, STRIPE)])
        if with_deg:
            pltpu.sync_copy(deg_sh.at[pl.ds(r0, STRIPE)], degb)
            pltpu.sync_copy(degb, out_deg.at[pl.ds(base + r0, STRIPE)])
        plsc.subcore_barrier()


def _make_spmv(with_deg):
    mesh = plsc.VectorSubcoreMesh(core_axis_name="c", subcore_axis_name="s",
                                  num_cores=2, num_subcores=16)
    out_type = [jax.ShapeDtypeStruct((NPAD, D), jnp.float32)]
    if with_deg:
        out_type.append(jax.ShapeDtypeStruct((NPAD,), jnp.float32))
    scratch = [
        pltpu.VMEM((MW,), jnp.int32),
        pltpu.VMEM((MW,), jnp.int32),
        pltpu.VMEM((BLK,), jnp.float32),
        pltpu.VMEM((BLK,), jnp.float32),
        pltpu.VMEM((BLK,), jnp.int32),
        pltpu.VMEM((BLK,), jnp.int32),
        pltpu.VMEM((BLK,), jnp.float32),
        pltpu.VMEM((BLK,), jnp.float32),
    ]
    if with_deg:
        scratch.append(pltpu.VMEM((BLK,), jnp.float32))
        scratch.append(pltpu.VMEM((BLK,), jnp.float32))
    scratch.append(pltpu.VMEM((BLK, D), jnp.float32))
    scratch.append(pltpu.VMEM((BLK, D), jnp.float32))
    scratch.append(pltpu.VMEM((16, D), jnp.float32))
    if with_deg:
        scratch.append(pltpu.VMEM((STRIPE,), jnp.float32))
        scratch.append(pltpu.VMEM((STRIPE,), jnp.float32))
    scratch.append(pltpu.VMEM_SHARED((Q, D), jnp.float32))
    if with_deg:
        scratch.append(pltpu.VMEM_SHARED((Q,), jnp.float32))
    scratch.append(pltpu.SemaphoreType.DMA)
    scratch.append(pltpu.SemaphoreType.DMA)
    scratch.append(pltpu.SemaphoreType.DMA)
    scratch.append(pltpu.SemaphoreType.DMA)
    if with_deg:
        scratch.append(pltpu.SemaphoreType.DMA)
        scratch.append(pltpu.SemaphoreType.DMA)

    return pl.kernel(
        functools.partial(_spmv_body, with_deg),
        out_type=tuple(out_type) if with_deg else out_type[0],
        mesh=mesh,
        scratch_types=scratch,
    )


def _pack_meta(srcv, dstv):
    return jnp.stack([srcv.reshape(NBLK, BLK), dstv.reshape(NBLK, BLK)],
                     axis=1).reshape(-1)


BR = 2000


def _combine2_body(relu, mlp, x, n1, d1, n2, d2, *rest):
    if mlp:
        (wsc, wn1, wn2, bb,
         l1w, l1b, l2w, l2b, l3w, l3b, out) = rest
    else:
        wsc, wn1, wn2, bb, out = rest
    r1 = 1.0 / jnp.maximum(d1[...], 1.0)
    r2 = 1.0 / jnp.maximum(d2[...], 1.0)
    acc = jnp.dot(x[...], wsc[...], preferred_element_type=jnp.float32)
    acc += jnp.dot(n1[...] * r1, wn1[...], preferred_element_type=jnp.float32)
    acc += jnp.dot(n2[...] * r2, wn2[...], preferred_element_type=jnp.float32)
    acc += bb[...]
    if relu:
        acc = jnp.maximum(acc, 0.0)
    if mlp:
        h = jnp.maximum(jnp.dot(acc, l1w[...],
                                preferred_element_type=jnp.float32) + l1b[...],
                        0.0)
        h = jnp.maximum(jnp.dot(h, l2w[...],
                                preferred_element_type=jnp.float32) + l2b[...],
                        0.0)
        out[...] = jnp.dot(h, l3w[...],
                           preferred_element_type=jnp.float32) + l3b[...]
    else:
        out[...] = acc


def _combine1_body(relu, x, nn, d, wsc, wn, bb, out):
    r = 1.0 / jnp.maximum(d[...], 1.0)
    acc = jnp.dot(x[...], wsc[...], preferred_element_type=jnp.float32)
    acc += jnp.dot(nn[...] * r, wn[...], preferred_element_type=jnp.float32)
    acc += bb[...]
    if relu:
        acc = jnp.maximum(acc, 0.0)
    out[...] = acc


def _row_spec(w):
    return pl.BlockSpec((BR, w), lambda i: (i, 0))


def _whole_spec(shape):
    return pl.BlockSpec(shape, lambda i: tuple(0 for _ in shape))


def _combine2(relu, mlp, x, n1, d1, n2, d2, wsc, wn1, wn2, bb, lws=None):
    grid = N // BR
    in_specs = [_row_spec(D), _row_spec(D), _row_spec(1),
                _row_spec(D), _row_spec(1),
                _whole_spec((D, D)), _whole_spec((D, D)), _whole_spec((D, D)),
                _whole_spec((1, D))]
    args = [x, n1, d1, n2, d2, wsc, wn1, wn2, bb]
    if mlp:
        l1w, l1b, l2w, l2b, l3w, l3b = lws
        in_specs += [_whole_spec((D, 64)), _whole_spec((1, 64)),
                     _whole_spec((64, 32)), _whole_spec((1, 32)),
                     _whole_spec((32, 4)), _whole_spec((1, 4))]
        args += [l1w, l1b, l2w, l2b, l3w, l3b]
        out_shape = jax.ShapeDtypeStruct((N, 4), jnp.float32)
        out_specs = _row_spec(4)
    else:
        out_shape = jax.ShapeDtypeStruct((N, D), jnp.float32)
        out_specs = _row_spec(D)
    return pl.pallas_call(
        functools.partial(_combine2_body, relu, mlp),
        grid=(grid,),
        in_specs=in_specs,
        out_specs=out_specs,
        out_shape=out_shape,
    )(*args)


def _combine1(relu, x, nn, d, wsc, wn, bb):
    grid = N // BR
    in_specs = [_row_spec(D), _row_spec(D), _row_spec(1),
                _whole_spec((D, D)), _whole_spec((D, D)), _whole_spec((1, D))]
    return pl.pallas_call(
        functools.partial(_combine1_body, relu),
        grid=(grid,),
        in_specs=in_specs,
        out_specs=_row_spec(D),
        out_shape=jax.ShapeDtypeStruct((N, D), jnp.float32),
    )(x, nn, d, wsc, wn, bb)


def kernel(x_acoustic, x_word, ei_sim_tic, ei_sim_w, ei_related_to,
           ew_sim_tic, ew_sim_w, ew_related_to,
           c1_tic_Ws, c1_tic_Wn, c1_tic_b, c1_w_Ws, c1_w_Wn, c1_w_b,
           c1_rel_Ws, c1_rel_Wn, c1_rel_b,
           c2_tic_Ws, c2_tic_Wn, c2_tic_b, c2_w_Ws, c2_w_Wn, c2_w_b,
           c2_rel_Ws, c2_rel_Wn, c2_rel_b,
           l1_W, l1_b, l2_W, l2_b, l3_W, l3_b):
    spmv_deg = _make_spmv(True)
    spmv = _make_spmv(False)

    meta_t = _pack_meta(ei_sim_tic[0], ei_sim_tic[1])
    meta_r = _pack_meta(ei_related_to[0], ei_related_to[1])
    meta_w = _pack_meta(ei_sim_w[0], ei_sim_w[1])

    a1t, degt = spmv_deg(x_acoustic, meta_t, ew_sim_tic)
    a1r, degr = spmv_deg(x_acoustic, meta_r, ew_related_to)
    a1w, degw = spmv_deg(x_word, meta_w, ew_sim_w)

    d_t = degt[:N, None]
    d_r = degr[:N, None]
    d_w = degw[:N, None]

    wsc1 = (c1_tic_Ws + c1_rel_Ws) * 0.5
    b1 = ((c1_tic_b + c1_rel_b) * 0.5)[None, :]
    a_full = _combine2(True, False,
                       x_acoustic, a1t[:N], d_t, a1r[:N], d_r,
                       wsc1, c1_tic_Wn * 0.5, c1_rel_Wn * 0.5, b1)

    w_full = _combine1(True, x_word, a1w[:N], d_w,
                       c1_w_Ws, c1_w_Wn, c1_w_b[None, :])

    a2t = spmv(a_full, meta_t, ew_sim_tic)
    a2r = spmv(a_full, meta_r, ew_related_to)
    a2w = spmv(w_full, meta_w, ew_sim_w)

    wsc2 = (c2_tic_Ws + c2_rel_Ws) * 0.5
    b2 = ((c2_tic_b + c2_rel_b) * 0.5)[None, :]
    h = _combine2(False, True,
                  a_full, a2t[:N], d_t, a2r[:N], d_r,
                  wsc2, c2_tic_Wn * 0.5, c2_rel_Wn * 0.5, b2,
                  lws=(l1_W, l1_b[None, :], l2_W, l2_b[None, :],
                       l3_W, l3_b[None, :]))

    w2 = _combine1(False, w_full, a2w[:N], d_w,
                   c2_w_Ws, c2_w_Wn, c2_w_b[None, :])

    return h, h, w2
